# baseline (speedup 1.0000x reference)
import functools

import jax
import jax.numpy as jnp
from jax import lax
from jax.experimental import pallas as pl
from jax.experimental.pallas import tpu as pltpu
from jax.experimental.pallas import tpu_sc as plsc

NN = 10000
EE = 320000
DIN = 128
DH = 256
CC = 64

NC = 2
NS = 16
NW = NC * NS
L = 16

CH = 80
CPW = 125
NP = 10000
STRIPE = NP // NS
D1 = DIN + 16
D2 = CC + 16

_mesh = plsc.VectorSubcoreMesh(core_axis_name="c", subcore_axis_name="s")
_sc_params = pltpu.CompilerParams(use_tc_tiling_on_sc=False,
                                  needs_layout_passes=False)


@functools.partial(
    pl.kernel,
    out_type=jax.ShapeDtypeStruct((NC, NP, D1), jnp.float32),
    mesh=_mesh,
    scratch_types=[
        pltpu.VMEM((CPW, CH), jnp.int32),
        pltpu.VMEM((CPW, CH), jnp.int32),
        pltpu.VMEM((CH, D1), jnp.float32),
        pltpu.VMEM_SHARED((NP, D1), jnp.float32),
        pltpu.SemaphoreType.DMA,
    ],
    compiler_params=_sc_params,
)
def _sc_aggregate(xe_hbm, srcp_hbm, dstp_hbm, z_hbm, a_hbm,
                  src_v, dst_v, gbuf, acc, sem):
    c = lax.axis_index("c")
    s = lax.axis_index("s")
    w = c * NS + s
    pltpu.sync_copy(srcp_hbm.at[w], src_v)
    pltpu.sync_copy(dstp_hbm.at[w], dst_v)
    pltpu.sync_copy(z_hbm, acc.at[pl.ds(s * STRIPE, STRIPE)])
    plsc.subcore_barrier()

    @pl.loop(0, CPW)
    def _(j):
        pltpu.async_copy(xe_hbm.at[src_v.at[j]], gbuf, sem).wait()
        pltpu.sync_copy(gbuf, acc.at[dst_v.at[j]], add=True)

    plsc.subcore_barrier()
    pltpu.sync_copy(acc.at[pl.ds(s * STRIPE, STRIPE)],
                    a_hbm.at[c, pl.ds(s * STRIPE, STRIPE)])


@functools.partial(
    pl.kernel,
    out_type=jax.ShapeDtypeStruct((NC, NP, D2), jnp.float32),
    mesh=_mesh,
    scratch_types=[
        pltpu.VMEM((CPW, CH), jnp.int32),
        pltpu.VMEM((CPW, CH), jnp.int32),
        pltpu.VMEM((NP,), jnp.float32),
        pltpu.VMEM((NP,), jnp.float32),
        pltpu.VMEM((CH, CC), jnp.float32),
        pltpu.VMEM((CH, D2), jnp.float32),
        pltpu.VMEM((CH,), jnp.float32),
        pltpu.VMEM_SHARED((NP, D2), jnp.float32),
        pltpu.SemaphoreType.DMA,
    ],
    compiler_params=_sc_params,
)
def _sc_attention(feat_hbm, el_hbm, er_hbm, srcp_hbm, dstp_hbm, z_hbm, w_hbm,
                  src_v, dst_v, el_v, er_v, fbuf, obuf, eebuf, acc, sem):
    c = lax.axis_index("c")
    s = lax.axis_index("s")
    w = c * NS + s
    pltpu.sync_copy(srcp_hbm.at[w], src_v)
    pltpu.sync_copy(dstp_hbm.at[w], dst_v)
    pltpu.sync_copy(el_hbm, el_v)
    pltpu.sync_copy(er_hbm, er_v)
    pltpu.sync_copy(z_hbm, acc.at[pl.ds(s * STRIPE, STRIPE)])
    plsc.subcore_barrier()

    @pl.loop(0, CPW)
    def _(j):
        cp = pltpu.async_copy(feat_hbm.at[src_v.at[j]], fbuf, sem)

        @pl.loop(0, CH // L)
        def _(g):
            sv = src_v[j, pl.ds(g * L, L)]
            dv = dst_v[j, pl.ds(g * L, L)]
            z = plsc.load_gather(el_v, [sv]) + plsc.load_gather(er_v, [dv])
            e = jnp.where(z < 0.0, z * jnp.float32(0.2), z)
            eebuf[pl.ds(g * L, L)] = jnp.exp(e)

        cp.wait()

        @pl.loop(0, CH // L)
        def _(g):
            eev = eebuf[pl.ds(g * L, L)]
            for k in range(L):
                v = eev.at[jnp.full((L,), k, jnp.int32)].get(
                    mode="promise_in_bounds")
                i = g * L + k
                obuf[i, pl.ds(CC, L)] = v
                for q in range(CC // L):
                    obuf[i, pl.ds(q * L, L)] = fbuf[i, pl.ds(q * L, L)] * v

        pltpu.sync_copy(obuf, acc.at[dst_v.at[j]], add=True)

    plsc.subcore_barrier()
    pltpu.sync_copy(acc.at[pl.ds(s * STRIPE, STRIPE)],
                    w_hbm.at[c, pl.ds(s * STRIPE, STRIPE)])


BN = 2000
_HI = jax.lax.Precision.HIGHEST


def _tc1_body(x_ref, a_ref, ws_ref, wn_ref, b0_ref, wg_ref, al_ref, ar_ref,
              feat_ref, el_ref, er_ref):
    a = a_ref[...]
    agg = (a[0, :, :DIN] + a[1, :, :DIN]) \
        / jnp.maximum(a[0, :, DIN] + a[1, :, DIN], 1.0)[:, None]
    h = (jnp.dot(x_ref[...], ws_ref[...], preferred_element_type=jnp.float32,
                 precision=_HI)
         + jnp.dot(agg, wn_ref[...], preferred_element_type=jnp.float32,
                   precision=_HI)
         + b0_ref[...])
    h = jnp.where(h > 0.0, h, jnp.exp(h) - 1.0)
    feat = jnp.dot(h, wg_ref[...], preferred_element_type=jnp.float32,
                   precision=_HI)
    feat_ref[...] = feat
    el_ref[...] = jnp.sum(feat * al_ref[...], axis=1, keepdims=True)
    er_ref[...] = jnp.sum(feat * ar_ref[...], axis=1, keepdims=True)


_tc1 = pl.pallas_call(
    _tc1_body,
    grid=(NN // BN,),
    in_specs=[
        pl.BlockSpec((BN, DIN), lambda i: (i, 0)),
        pl.BlockSpec((NC, BN, D1), lambda i: (0, i, 0)),
        pl.BlockSpec((DIN, DH), lambda i: (0, 0)),
        pl.BlockSpec((DIN, DH), lambda i: (0, 0)),
        pl.BlockSpec((DH,), lambda i: (0,)),
        pl.BlockSpec((DH, CC), lambda i: (0, 0)),
        pl.BlockSpec((CC,), lambda i: (0,)),
        pl.BlockSpec((CC,), lambda i: (0,)),
    ],
    out_specs=[
        pl.BlockSpec((BN, CC), lambda i: (i, 0)),
        pl.BlockSpec((BN, 1), lambda i: (i, 0)),
        pl.BlockSpec((BN, 1), lambda i: (i, 0)),
    ],
    out_shape=[
        jax.ShapeDtypeStruct((NN, CC), jnp.float32),
        jax.ShapeDtypeStruct((NN, 1), jnp.float32),
        jax.ShapeDtypeStruct((NN, 1), jnp.float32),
    ],
)


def _tc2_body(w_ref, bg_ref, out_ref):
    wp = w_ref[...]
    num = wp[0, :, :CC] + wp[1, :, :CC]
    den = jnp.maximum(wp[0, :, CC] + wp[1, :, CC], 1e-9)
    out_ref[...] = num / den[:, None] + bg_ref[...]


_tc2 = pl.pallas_call(
    _tc2_body,
    grid=(NN // BN,),
    in_specs=[
        pl.BlockSpec((NC, BN, D2), lambda i: (0, i, 0)),
        pl.BlockSpec((CC,), lambda i: (0,)),
    ],
    out_specs=pl.BlockSpec((BN, CC), lambda i: (i, 0)),
    out_shape=jax.ShapeDtypeStruct((NN, CC), jnp.float32),
)


def kernel(x, edge_index, W_self, W_neigh, b0, W_gat, attn_l, attn_r, b_gat):
    src = edge_index[0]
    dst = edge_index[1]
    srcp = src.reshape(NW, CPW, CH)
    dstp = dst.reshape(NW, CPW, CH)
    x_ext = jnp.concatenate([x, jnp.ones((NN, 16), jnp.float32)], axis=1)
    z1 = jnp.zeros((STRIPE, D1), jnp.float32)
    z2 = jnp.zeros((STRIPE, D2), jnp.float32)

    a_part = _sc_aggregate(x_ext, srcp, dstp, z1)
    feat, el, er = _tc1(x, a_part, W_self, W_neigh, b0, W_gat, attn_l, attn_r)
    w_part = _sc_attention(feat, el.reshape(NN), er.reshape(NN), srcp, dstp,
                           z2)
    return _tc2(w_part, b_gat)


# speedup vs baseline: 1.2784x; 1.2784x over previous
import functools

import jax
import jax.numpy as jnp
from jax import lax
from jax.experimental import pallas as pl
from jax.experimental.pallas import tpu as pltpu
from jax.experimental.pallas import tpu_sc as plsc

NN = 10000
EE = 320000
DIN = 128
DH = 256
CC = 64

NC = 2
NS = 16
NW = NC * NS
L = 16

CH = 80
CPW = 125
NP = 10000
STRIPE = NP // NS
D1 = DIN + 16
D2 = CC + 16

_mesh = plsc.VectorSubcoreMesh(core_axis_name="c", subcore_axis_name="s")
_sc_params = pltpu.CompilerParams(use_tc_tiling_on_sc=False,
                                  needs_layout_passes=False)


@functools.partial(
    pl.kernel,
    out_type=jax.ShapeDtypeStruct((NC, NP, D1), jnp.float32),
    mesh=_mesh,
    scratch_types=[
        pltpu.VMEM((CPW, CH), jnp.int32),
        pltpu.VMEM((CPW, CH), jnp.int32),
        pltpu.VMEM((CH, D1), jnp.float32),
        pltpu.VMEM_SHARED((NP, D1), jnp.float32),
        pltpu.SemaphoreType.DMA,
    ],
    compiler_params=_sc_params,
)
def _sc_aggregate(xe_hbm, srcp_hbm, dstp_hbm, z_hbm, a_hbm,
                  src_v, dst_v, gbuf, acc, sem):
    c = lax.axis_index("c")
    s = lax.axis_index("s")
    w = c * NS + s
    pltpu.sync_copy(srcp_hbm.at[w], src_v)
    pltpu.sync_copy(dstp_hbm.at[w], dst_v)
    pltpu.sync_copy(z_hbm, acc.at[pl.ds(s * STRIPE, STRIPE)])
    plsc.subcore_barrier()

    @pl.loop(0, CPW)
    def _(j):
        pltpu.async_copy(xe_hbm.at[src_v.at[j]], gbuf, sem).wait()
        pltpu.sync_copy(gbuf, acc.at[dst_v.at[j]], add=True)

    plsc.subcore_barrier()
    pltpu.sync_copy(acc.at[pl.ds(s * STRIPE, STRIPE)],
                    a_hbm.at[c, pl.ds(s * STRIPE, STRIPE)])


@functools.partial(
    pl.kernel,
    out_type=jax.ShapeDtypeStruct((NC, NP, D2), jnp.float32),
    mesh=_mesh,
    scratch_types=[
        pltpu.VMEM((CPW, CH), jnp.int32),
        pltpu.VMEM((CPW, CH), jnp.int32),
        pltpu.VMEM((NP,), jnp.float32),
        pltpu.VMEM((NP,), jnp.float32),
        pltpu.VMEM((CH, CC), jnp.float32),
        pltpu.VMEM((CH, D2), jnp.float32),
        pltpu.VMEM((CH,), jnp.float32),
        pltpu.VMEM_SHARED((NP, D2), jnp.float32),
        pltpu.SemaphoreType.DMA,
    ],
    compiler_params=_sc_params,
)
def _sc_attention(feat_hbm, el_hbm, er_hbm, srcp_hbm, dstp_hbm, z_hbm, w_hbm,
                  src_v, dst_v, el_v, er_v, fbuf, obuf, eebuf, acc, sem):
    c = lax.axis_index("c")
    s = lax.axis_index("s")
    w = c * NS + s
    pltpu.sync_copy(srcp_hbm.at[w], src_v)
    pltpu.sync_copy(dstp_hbm.at[w], dst_v)
    pltpu.sync_copy(el_hbm, el_v)
    pltpu.sync_copy(er_hbm, er_v)
    pltpu.sync_copy(z_hbm, acc.at[pl.ds(s * STRIPE, STRIPE)])
    plsc.subcore_barrier()

    @pl.loop(0, CPW)
    def _(j):
        cp = pltpu.async_copy(feat_hbm.at[src_v.at[j]], fbuf, sem)

        for g in range(CH // L):
            sv = src_v[j, pl.ds(g * L, L)]
            dv = dst_v[j, pl.ds(g * L, L)]
            z = plsc.load_gather(el_v, [sv]) + plsc.load_gather(er_v, [dv])
            e = jnp.where(z < 0.0, z * jnp.float32(0.2), z)
            eebuf[pl.ds(g * L, L)] = jnp.exp(e)

        cp.wait()

        for g in range(CH // L):
            eev = eebuf[pl.ds(g * L, L)]
            for k in range(L):
                v = eev.at[jnp.full((L,), k, jnp.int32)].get(
                    mode="promise_in_bounds")
                i = g * L + k
                obuf[i, pl.ds(CC, L)] = v
                for q in range(CC // L):
                    obuf[i, pl.ds(q * L, L)] = fbuf[i, pl.ds(q * L, L)] * v

        pltpu.sync_copy(obuf, acc.at[dst_v.at[j]], add=True)

    plsc.subcore_barrier()
    pltpu.sync_copy(acc.at[pl.ds(s * STRIPE, STRIPE)],
                    w_hbm.at[c, pl.ds(s * STRIPE, STRIPE)])


BN = 2000
_HI = jax.lax.Precision.HIGHEST


def _tc1_body(x_ref, a_ref, ws_ref, wn_ref, b0_ref, wg_ref, al_ref, ar_ref,
              feat_ref, el_ref, er_ref):
    a = a_ref[...]
    agg = (a[0, :, :DIN] + a[1, :, :DIN]) \
        / jnp.maximum(a[0, :, DIN] + a[1, :, DIN], 1.0)[:, None]
    h = (jnp.dot(x_ref[...], ws_ref[...], preferred_element_type=jnp.float32,
                 precision=_HI)
         + jnp.dot(agg, wn_ref[...], preferred_element_type=jnp.float32,
                   precision=_HI)
         + b0_ref[...])
    h = jnp.where(h > 0.0, h, jnp.exp(h) - 1.0)
    feat = jnp.dot(h, wg_ref[...], preferred_element_type=jnp.float32,
                   precision=_HI)
    feat_ref[...] = feat
    el_ref[...] = jnp.sum(feat * al_ref[...], axis=1, keepdims=True)
    er_ref[...] = jnp.sum(feat * ar_ref[...], axis=1, keepdims=True)


_tc1 = pl.pallas_call(
    _tc1_body,
    grid=(NN // BN,),
    in_specs=[
        pl.BlockSpec((BN, DIN), lambda i: (i, 0)),
        pl.BlockSpec((NC, BN, D1), lambda i: (0, i, 0)),
        pl.BlockSpec((DIN, DH), lambda i: (0, 0)),
        pl.BlockSpec((DIN, DH), lambda i: (0, 0)),
        pl.BlockSpec((DH,), lambda i: (0,)),
        pl.BlockSpec((DH, CC), lambda i: (0, 0)),
        pl.BlockSpec((CC,), lambda i: (0,)),
        pl.BlockSpec((CC,), lambda i: (0,)),
    ],
    out_specs=[
        pl.BlockSpec((BN, CC), lambda i: (i, 0)),
        pl.BlockSpec((BN, 1), lambda i: (i, 0)),
        pl.BlockSpec((BN, 1), lambda i: (i, 0)),
    ],
    out_shape=[
        jax.ShapeDtypeStruct((NN, CC), jnp.float32),
        jax.ShapeDtypeStruct((NN, 1), jnp.float32),
        jax.ShapeDtypeStruct((NN, 1), jnp.float32),
    ],
)


def _tc2_body(w_ref, bg_ref, out_ref):
    wp = w_ref[...]
    num = wp[0, :, :CC] + wp[1, :, :CC]
    den = jnp.maximum(wp[0, :, CC] + wp[1, :, CC], 1e-9)
    out_ref[...] = num / den[:, None] + bg_ref[...]


_tc2 = pl.pallas_call(
    _tc2_body,
    grid=(NN // BN,),
    in_specs=[
        pl.BlockSpec((NC, BN, D2), lambda i: (0, i, 0)),
        pl.BlockSpec((CC,), lambda i: (0,)),
    ],
    out_specs=pl.BlockSpec((BN, CC), lambda i: (i, 0)),
    out_shape=jax.ShapeDtypeStruct((NN, CC), jnp.float32),
)


def kernel(x, edge_index, W_self, W_neigh, b0, W_gat, attn_l, attn_r, b_gat):
    src = edge_index[0]
    dst = edge_index[1]
    srcp = src.reshape(NW, CPW, CH)
    dstp = dst.reshape(NW, CPW, CH)
    x_ext = jnp.concatenate([x, jnp.ones((NN, 16), jnp.float32)], axis=1)
    z1 = jnp.zeros((STRIPE, D1), jnp.float32)
    z2 = jnp.zeros((STRIPE, D2), jnp.float32)

    a_part = _sc_aggregate(x_ext, srcp, dstp, z1)
    feat, el, er = _tc1(x, a_part, W_self, W_neigh, b0, W_gat, attn_l, attn_r)
    w_part = _sc_attention(feat, el.reshape(NN), er.reshape(NN), srcp, dstp,
                           z2)
    return _tc2(w_part, b_gat)
